# baseline (speedup 1.0000x reference)
import functools

import jax
import jax.numpy as jnp
from jax import lax
from jax.experimental import pallas as pl
from jax.experimental.pallas import tpu as pltpu
from jax.experimental.pallas import tpu_sc as plsc

_D = 64
_B = 4096 * 200
_NW = 32
_BLK = 128
_NBLK = _B // _BLK
_BLK_PER_W = _NBLK // _NW
_K = 8
_NITER = _BLK_PER_W // _K


@functools.partial(
    pl.kernel,
    mesh=plsc.VectorSubcoreMesh(core_axis_name="c", subcore_axis_name="s"),
    out_type=jax.ShapeDtypeStruct((_NBLK, _BLK, _D), jnp.float32),
    scratch_types=[
        pltpu.VMEM((_K, _BLK), jnp.int32),
        pltpu.VMEM((_K, _BLK, _D), jnp.float32),
        pltpu.SemaphoreType.DMA,
    ],
    compiler_params=pltpu.CompilerParams(use_tc_tiling_on_sc=False),
)
def _embed_gather(idx_hbm, table_hbm, out_hbm, idx_v, rows_v, sem):
    wid = lax.axis_index("s") * 2 + lax.axis_index("c")
    base = wid * _BLK_PER_W

    def body(i, carry):
        row = base + i * _K
        pltpu.sync_copy(idx_hbm.at[pl.ds(row, _K)], idx_v)
        copies = [
            pltpu.async_copy(table_hbm.at[idx_v.at[j]], rows_v.at[j], sem)
            for j in range(_K)
        ]
        for c in copies:
            c.wait()
        pltpu.sync_copy(rows_v, out_hbm.at[pl.ds(row, _K)])
        return carry

    lax.fori_loop(0, _NITER, body, 0)


def kernel(input_sentence, table):
    idx = input_sentence.reshape(_NBLK, _BLK).astype(jnp.int32)
    out = _embed_gather(idx, table)
    return out.reshape(input_sentence.shape + (_D,))


# speedup vs baseline: 1.0160x; 1.0160x over previous
import functools

import jax
import jax.numpy as jnp
from jax import lax
from jax.experimental import pallas as pl
from jax.experimental.pallas import tpu as pltpu
from jax.experimental.pallas import tpu_sc as plsc

_D = 64
_B = 4096 * 200
_NW = 32
_BLK = 128
_NBLK = _B // _BLK
_BLK_PER_W = _NBLK // _NW
_K = 5
_NITER = _BLK_PER_W // _K
_NITER2 = _NITER // 2


@functools.partial(
    pl.kernel,
    mesh=plsc.VectorSubcoreMesh(core_axis_name="c", subcore_axis_name="s"),
    out_type=jax.ShapeDtypeStruct((_NBLK, _BLK, _D), jnp.float32),
    scratch_types=[
        pltpu.VMEM((2, _K, _BLK), jnp.int32),
        pltpu.VMEM((2, _K, _BLK, _D), jnp.float32),
        pltpu.SemaphoreType.DMA,
        pltpu.SemaphoreType.DMA,
        pltpu.SemaphoreType.DMA,
        pltpu.SemaphoreType.DMA,
        pltpu.SemaphoreType.DMA,
    ],
    compiler_params=pltpu.CompilerParams(use_tc_tiling_on_sc=False),
)
def _embed_gather(idx_hbm, table_hbm, out_hbm, idx_v, rows_v, sem_i0, sem_i1,
                  sem_w0, sem_w1, sem_g):
    wid = lax.axis_index("s") * 2 + lax.axis_index("c")
    base = wid * _BLK_PER_W
    sem_i = (sem_i0, sem_i1)
    sem_w = (sem_w0, sem_w1)

    def start_idx(chunk, buf):
        return pltpu.async_copy(
            idx_hbm.at[pl.ds(base + chunk * _K, _K)], idx_v.at[buf], sem_i[buf])

    def fire_gathers(buf):
        return [
            pltpu.async_copy(
                table_hbm.at[idx_v.at[buf].at[j]], rows_v.at[buf].at[j], sem_g)
            for j in range(_K)
        ]

    def start_write(chunk, buf):
        return pltpu.async_copy(
            rows_v.at[buf], out_hbm.at[pl.ds(base + chunk * _K, _K)], sem_w[buf])

    start_idx(0, 0).wait()
    g = fire_gathers(0)
    start_idx(1, 1)
    for c in g:
        c.wait()
    start_write(0, 0)

    pltpu.make_async_copy(
        idx_hbm.at[pl.ds(base, _K)], idx_v.at[1], sem_i[1]).wait()
    g = fire_gathers(1)
    start_idx(2, 0)
    for c in g:
        c.wait()
    start_write(1, 1)

    def body(gi, carry):
        for b in (0, 1):
            chunk = 2 * gi + b
            row = base + chunk * _K
            pltpu.make_async_copy(
                idx_hbm.at[pl.ds(row, _K)], idx_v.at[b], sem_i[b]).wait()
            pltpu.make_async_copy(
                rows_v.at[b], out_hbm.at[pl.ds(row, _K)], sem_w[b]).wait()
            g = fire_gathers(b)
            nxt = lax.rem(chunk + 1, _NITER)
            start_idx(nxt, 1 - b)
            for c in g:
                c.wait()
            start_write(chunk, b)
        return carry

    lax.fori_loop(1, _NITER2, body, 0)

    pltpu.make_async_copy(
        idx_hbm.at[pl.ds(base, _K)], idx_v.at[0], sem_i[0]).wait()
    pltpu.make_async_copy(
        rows_v.at[0], out_hbm.at[pl.ds(base, _K)], sem_w[0]).wait()
    pltpu.make_async_copy(
        rows_v.at[1], out_hbm.at[pl.ds(base, _K)], sem_w[1]).wait()


def kernel(input_sentence, table):
    idx = input_sentence.reshape(_NBLK, _BLK).astype(jnp.int32)
    out = _embed_gather(idx, table)
    return out.reshape(input_sentence.shape + (_D,))


# speedup vs baseline: 1.2678x; 1.2477x over previous
import functools

import jax
import jax.numpy as jnp
from jax import lax
from jax.experimental import pallas as pl
from jax.experimental.pallas import tpu as pltpu
from jax.experimental.pallas import tpu_sc as plsc

_D = 64
_B = 4096 * 200
_NW = 32
_BLK = 128
_NBLK = _B // _BLK
_BLK_PER_W = _NBLK // _NW
_K = 5
_NITER = _BLK_PER_W // _K
_NITER2 = _NITER // 2


@functools.partial(
    pl.kernel,
    mesh=plsc.VectorSubcoreMesh(core_axis_name="c", subcore_axis_name="s"),
    out_type=jax.ShapeDtypeStruct((_B, _D), jnp.float32),
    scratch_types=[
        pltpu.VMEM((2, _K, _BLK), jnp.int32),
        pltpu.VMEM((2, _K * _BLK, _D), jnp.float32),
        pltpu.SemaphoreType.DMA,
        pltpu.SemaphoreType.DMA,
        pltpu.SemaphoreType.DMA,
        pltpu.SemaphoreType.DMA,
        pltpu.SemaphoreType.DMA,
    ],
    compiler_params=pltpu.CompilerParams(use_tc_tiling_on_sc=False),
)
def _embed_gather(idx_hbm, table_hbm, out_hbm, idx_v, rows_v, sem_i0, sem_i1,
                  sem_w0, sem_w1, sem_g):
    wid = lax.axis_index("s") * 2 + lax.axis_index("c")
    base = wid * _BLK_PER_W
    rbase = wid * _BLK_PER_W * _BLK
    _CR = _K * _BLK
    sem_i = (sem_i0, sem_i1)
    sem_w = (sem_w0, sem_w1)

    def start_idx(chunk, buf):
        return pltpu.async_copy(
            idx_hbm.at[pl.ds(base + chunk * _K, _K)], idx_v.at[buf], sem_i[buf])

    def fire_gathers(buf):
        return [
            pltpu.async_copy(
                table_hbm.at[idx_v.at[buf].at[j]],
                rows_v.at[buf].at[pl.ds(j * _BLK, _BLK)], sem_g)
            for j in range(_K)
        ]

    def start_write(chunk, buf):
        return pltpu.async_copy(
            rows_v.at[buf], out_hbm.at[pl.ds(rbase + chunk * _CR, _CR)],
            sem_w[buf])

    start_idx(0, 0).wait()
    g = fire_gathers(0)
    start_idx(1, 1)
    for c in g:
        c.wait()
    start_write(0, 0)

    pltpu.make_async_copy(
        idx_hbm.at[pl.ds(base, _K)], idx_v.at[1], sem_i[1]).wait()
    g = fire_gathers(1)
    start_idx(2, 0)
    for c in g:
        c.wait()
    start_write(1, 1)

    def body(gi, carry):
        for b in (0, 1):
            chunk = 2 * gi + b
            row = base + chunk * _K
            pltpu.make_async_copy(
                idx_hbm.at[pl.ds(row, _K)], idx_v.at[b], sem_i[b]).wait()
            pltpu.make_async_copy(
                rows_v.at[b], out_hbm.at[pl.ds(rbase, _CR)], sem_w[b]).wait()
            g = fire_gathers(b)
            nxt = lax.rem(chunk + 1, _NITER)
            start_idx(nxt, 1 - b)
            for c in g:
                c.wait()
            start_write(chunk, b)
        return carry

    lax.fori_loop(1, _NITER2, body, 0)

    pltpu.make_async_copy(
        idx_hbm.at[pl.ds(base, _K)], idx_v.at[0], sem_i[0]).wait()
    pltpu.make_async_copy(
        rows_v.at[0], out_hbm.at[pl.ds(rbase, _CR)], sem_w[0]).wait()
    pltpu.make_async_copy(
        rows_v.at[1], out_hbm.at[pl.ds(rbase, _CR)], sem_w[1]).wait()


def kernel(input_sentence, table):
    from jax.experimental.layout import Layout, with_layout_constraint
    idx = input_sentence.reshape(_NBLK, _BLK).astype(jnp.int32)
    tab = with_layout_constraint(
        table, Layout(major_to_minor=(0, 1), tiling=((8,),)))
    out = _embed_gather(idx, tab)
    return out.reshape(input_sentence.shape + (_D,))
